# baseline (speedup 1.0000x reference)
import functools

import jax
import jax.numpy as jnp
from jax import lax
from jax.experimental import pallas as pl
from jax.experimental.pallas import tpu as pltpu
from jax.experimental.pallas import tpu_sc as plsc

B, T, K = 64, 2048, 128
NC, NS = 2, 16
NW = NC * NS
ROWS_PER_W = B // NW
C = 128
NCH = T // C
L = 16
KV = K // L


def _sc_body(unaries_hbm, lengths_hbm, out_hbm, len_v, buf0, buf1, out_v,
             bvs_v, bis_v, sem0, sem1):
    cid = lax.axis_index("c")
    sid = lax.axis_index("s")
    wid = sid * NC + cid
    r0 = wid * ROWS_PER_W

    pltpu.sync_copy(lengths_hbm, len_v)
    iota0 = lax.iota(jnp.int32, L)
    lv = plsc.load_gather(len_v, [r0 + jnp.minimum(iota0, 1)])
    ln0 = jnp.minimum(jnp.maximum(lv[0], 0), T)
    ln1 = jnp.minimum(jnp.maximum(lv[1], 0), T)
    n0 = (ln0 + C - 1) // C
    n1 = (ln1 + C - 1) // C
    ntot = n0 + n1

    iota = lax.iota(jnp.int32, L)
    idx_c = [(K - 1 - j * L) - iota for j in range(KV)]
    col1 = iota * (L + 1)

    def chunk_src(j):
        in_r1 = (j >= n0).astype(jnp.int32)
        t0 = jnp.where(j < n0, j, j - n0) * C
        return r0 + in_r1, in_r1, t0

    def start(j, buf, sem):
        row, _, t0 = chunk_src(j)
        pltpu.async_copy(unaries_hbm.at[row, pl.ds(t0, C)], buf, sem)

    def wait(buf, sem):
        pltpu.make_async_copy(unaries_hbm.at[0, pl.ds(0, C)], buf, sem).wait()

    def combine(av, ai, bv, bi):
        m = bv > av
        return jnp.where(m, bv, av), jnp.where(m, bi, ai)

    def compute_chunk(j, buf):
        _, rloc, t0 = chunk_src(j)

        @plsc.parallel_loop(0, C // L, unroll=4)
        def grp(g):
            base = g * L
            sb = g * (L * (L + 1))
            for u in range(L):
                t = base + u
                vs = [buf[t, pl.ds(k * L, L)] for k in range(KV)]
                l1 = [combine(vs[2 * k], idx_c[2 * k], vs[2 * k + 1],
                              idx_c[2 * k + 1]) for k in range(4)]
                l2 = [combine(*l1[0], *l1[1]), combine(*l1[2], *l1[3])]
                bv, bi = combine(*l2[0], *l2[1])
                row_idx = iota + (sb + u * (L + 1))
                plsc.store_scatter(bvs_v, [row_idx], bv)
                plsc.store_scatter(bis_v, [row_idx], bi)
            cols_v = [plsc.load_gather(bvs_v, [col1 + (sb + p)])
                      for p in range(L)]
            cols_i = [plsc.load_gather(bis_v, [col1 + (sb + p)])
                      for p in range(L)]
            mx = cols_v
            while len(mx) > 1:
                mx = [jnp.maximum(mx[2 * i], mx[2 * i + 1])
                      for i in range(len(mx) // 2)]
            maxv = mx[0]
            cand = [jnp.where(cols_v[p] == maxv, cols_i[p], -1)
                    for p in range(L)]
            while len(cand) > 1:
                cand = [jnp.maximum(cand[2 * i], cand[2 * i + 1])
                        for i in range(len(cand) // 2)]
            out_v[rloc, pl.ds(t0 + base, L)] = (K - 1) - cand[0]

    @pl.when(ntot > 0)
    def _():
        start(0, buf0, sem0)

    def chunk_body(i, _):
        @pl.when(i % 2 == 0)
        def _():
            wait(buf0, sem0)

            @pl.when(i + 1 < ntot)
            def _():
                start(i + 1, buf1, sem1)

            compute_chunk(i, buf0)

        @pl.when(i % 2 == 1)
        def _():
            wait(buf1, sem1)

            @pl.when(i + 1 < ntot)
            def _():
                start(i + 1, buf0, sem0)

            compute_chunk(i, buf1)

        return 0

    lax.fori_loop(0, ntot, chunk_body, 0)

    for r, ln in ((0, ln0), (1, ln1)):
        def clean(g, _):
            tv = iota + g * L
            v = out_v[r, pl.ds(g * L, L)]
            out_v[r, pl.ds(g * L, L)] = jnp.where(tv < ln, v, 0)
            return 0

        lax.fori_loop(ln // L, T // L, clean, 0)

    pltpu.sync_copy(out_v, out_hbm.at[pl.ds(r0, ROWS_PER_W)])


@jax.jit
def kernel(unaries, lengths):
    mesh = plsc.VectorSubcoreMesh(core_axis_name="c", subcore_axis_name="s",
                                  num_cores=NC, num_subcores=NS)
    return pl.kernel(
        _sc_body,
        out_type=jax.ShapeDtypeStruct((B, T), jnp.int32),
        mesh=mesh,
        compiler_params=pltpu.CompilerParams(needs_layout_passes=False),
        scratch_types=[
            pltpu.VMEM((B,), jnp.int32),
            pltpu.VMEM((C, K), jnp.float32),
            pltpu.VMEM((C, K), jnp.float32),
            pltpu.VMEM((ROWS_PER_W, T), jnp.int32),
            pltpu.VMEM((C * (L + 1),), jnp.float32),
            pltpu.VMEM((C * (L + 1),), jnp.int32),
            pltpu.SemaphoreType.DMA,
            pltpu.SemaphoreType.DMA,
        ],
    )(unaries, lengths)


# speedup vs baseline: 1.3286x; 1.3286x over previous
import functools

import jax
import jax.numpy as jnp
from jax import lax
from jax.experimental import pallas as pl
from jax.experimental.pallas import tpu as pltpu
from jax.experimental.pallas import tpu_sc as plsc

B, T, K = 64, 2048, 128
NC, NS = 2, 16
NW = NC * NS
ROWS_PER_W = B // NW
C = 128
NCH = T // C
L = 16
KV = K // L


def _sc_body(unaries_hbm, lengths_hbm, out_hbm, len_v, buf2_v, out_v,
             bvs_v, bis_v, sem0, sem1):
    cid = lax.axis_index("c")
    sid = lax.axis_index("s")
    wid = sid * NC + cid
    r0 = wid * ROWS_PER_W

    pltpu.sync_copy(lengths_hbm, len_v)
    iota0 = lax.iota(jnp.int32, L)
    lv = plsc.load_gather(len_v, [r0 + jnp.minimum(iota0, 1)])
    ln0 = jnp.minimum(jnp.maximum(lv[0], 0), T)
    ln1 = jnp.minimum(jnp.maximum(lv[1], 0), T)
    n0 = (ln0 + C - 1) // C
    n1 = (ln1 + C - 1) // C
    ntot = n0 + n1

    iota = lax.iota(jnp.int32, L)
    idx_c = [(K - 1 - j * L) - iota for j in range(KV)]
    col1 = iota * (L + 1)

    def chunk_src(j):
        in_r1 = (j >= n0).astype(jnp.int32)
        t0 = jnp.where(j < n0, j, j - n0) * C
        return r0 + in_r1, in_r1, t0

    def start(j, p, sem):
        row, _, t0 = chunk_src(j)
        pltpu.async_copy(unaries_hbm.at[row, pl.ds(t0, C)], buf2_v.at[p], sem)

    def wait(p, sem):
        pltpu.make_async_copy(unaries_hbm.at[0, pl.ds(0, C)], buf2_v.at[p],
                              sem).wait()

    def combine(av, ai, bv, bi):
        m = bv > av
        return jnp.where(m, bv, av), jnp.where(m, bi, ai)

    def compute_chunk(j, par):
        _, rloc, t0 = chunk_src(j)

        @plsc.parallel_loop(0, C // L, unroll=2)
        def grp(g):
            base = g * L
            sb = g * (L * (L + 1))
            for u in range(L):
                t = base + u
                vs = [buf2_v[par, t, pl.ds(k * L, L)] for k in range(KV)]
                l1 = [combine(vs[2 * k], idx_c[2 * k], vs[2 * k + 1],
                              idx_c[2 * k + 1]) for k in range(4)]
                l2 = [combine(*l1[0], *l1[1]), combine(*l1[2], *l1[3])]
                bv, bi = combine(*l2[0], *l2[1])
                row_idx = iota + (sb + u * (L + 1))
                plsc.store_scatter(bvs_v, [row_idx], bv)
                plsc.store_scatter(bis_v, [row_idx], bi)
            cols_v = [plsc.load_gather(bvs_v, [col1 + (sb + p)])
                      for p in range(L)]
            cols_i = [plsc.load_gather(bis_v, [col1 + (sb + p)])
                      for p in range(L)]
            mx = cols_v
            while len(mx) > 1:
                mx = [jnp.maximum(mx[2 * i], mx[2 * i + 1])
                      for i in range(len(mx) // 2)]
            maxv = mx[0]
            cand = [jnp.where(cols_v[p] == maxv, cols_i[p], -1)
                    for p in range(L)]
            while len(cand) > 1:
                cand = [jnp.maximum(cand[2 * i], cand[2 * i + 1])
                        for i in range(len(cand) // 2)]
            out_v[rloc, pl.ds(t0 + base, L)] = (K - 1) - cand[0]

    @pl.when(ntot > 0)
    def _():
        start(0, 0, sem0)

    def chunk_body(i, _):
        par = i % 2

        @pl.when(par == 0)
        def _():
            wait(0, sem0)

            @pl.when(i + 1 < ntot)
            def _():
                start(i + 1, 1, sem1)

        @pl.when(par == 1)
        def _():
            wait(1, sem1)

            @pl.when(i + 1 < ntot)
            def _():
                start(i + 1, 0, sem0)

        compute_chunk(i, par)
        return 0

    lax.fori_loop(0, ntot, chunk_body, 0)

    for r, ln in ((0, ln0), (1, ln1)):
        def clean(g, _):
            tv = iota + g * L
            v = out_v[r, pl.ds(g * L, L)]
            out_v[r, pl.ds(g * L, L)] = jnp.where(tv < ln, v, 0)
            return 0

        lax.fori_loop(ln // L, T // L, clean, 0)

    pltpu.sync_copy(out_v, out_hbm.at[pl.ds(r0, ROWS_PER_W)])


@jax.jit
def kernel(unaries, lengths):
    mesh = plsc.VectorSubcoreMesh(core_axis_name="c", subcore_axis_name="s",
                                  num_cores=NC, num_subcores=NS)
    return pl.kernel(
        _sc_body,
        out_type=jax.ShapeDtypeStruct((B, T), jnp.int32),
        mesh=mesh,
        compiler_params=pltpu.CompilerParams(needs_layout_passes=False),
        scratch_types=[
            pltpu.VMEM((B,), jnp.int32),
            pltpu.VMEM((2, C, K), jnp.float32),
            pltpu.VMEM((ROWS_PER_W, T), jnp.int32),
            pltpu.VMEM((C * (L + 1),), jnp.float32),
            pltpu.VMEM((C * (L + 1),), jnp.int32),
            pltpu.SemaphoreType.DMA,
            pltpu.SemaphoreType.DMA,
        ],
    )(unaries, lengths)


# speedup vs baseline: 1.7899x; 1.3472x over previous
import jax
import jax.numpy as jnp
from jax import lax
from jax.experimental import pallas as pl
from jax.experimental.pallas import tpu as pltpu
from jax.experimental.pallas import tpu_sc as plsc

B, T, K = 64, 2048, 128
NC, NS = 2, 16
NW = NC * NS
C = 128
NCH = T // C
L = 16
KV = K // L
BIG = 2**31 - 1


def _sc_body(unaries_hbm, lengths_hbm, out_hbm, len_v, buf2_v, res_v, out_z,
             bvs_v, bis_v, pfx_v, sem_i0, sem_i1, sem_o0, sem_o1, sem_z):
    cid = lax.axis_index("c")
    sid = lax.axis_index("s")
    wid = sid * NC + cid
    r0 = wid * (B // NW)

    iota = lax.iota(jnp.int32, L)
    idx_c = [(K - 1 - j * L) - iota for j in range(KV)]
    col1 = iota * (L + 1)

    pltpu.sync_copy(lengths_hbm, len_v)
    zeros16 = jnp.zeros((L,), jnp.int32)
    for g in range(NCH // 2):
        out_z[pl.ds(g * L, L)] = zeros16
    carry = jnp.int32(0)
    nch_row = []
    for k in range(B // L):
        ln16 = len_v[pl.ds(k * L, L)]
        ln16 = jnp.minimum(jnp.maximum(ln16, 0), T)
        nb = (ln16 + (C - 1)) // C
        nch_row.append(nb)
        cum = plsc.cumsum(nb)
        pfx_v[pl.ds(k * L, L)] = (cum - nb) + carry
        carry = carry + cum[L - 1]
    ptot = carry
    pfx_v[pl.ds(B, L)] = jnp.where(iota == 0, ptot, BIG)
    pfx_v[pl.ds(B + L, L)] = jnp.full((L,), BIG, jnp.int32)

    def pgather(idx):
        return plsc.load_gather(pfx_v, [idx])

    def advance(b, g):
        def cond(bb):
            w16 = pgather(bb + 1 + iota) <= g
            return plsc.all_reduce_population_count(w16)[0] == L

        b = lax.while_loop(cond, lambda bb: bb + L, b)
        w16 = pgather(b + 1 + iota) <= g
        return b + plsc.all_reduce_population_count(w16)[0]

    def combine(av, ai, bv, bi):
        m = bv > av
        return jnp.where(m, bv, av), jnp.where(m, bi, ai)

    def compute_chunk(par, t0, lvb):
        @plsc.parallel_loop(0, C // L, unroll=2)
        def grp(g):
            base = g * L
            sb = g * (L * (L + 1))
            for u in range(L):
                t = base + u
                vs = [buf2_v[par, t, pl.ds(k * L, L)] for k in range(KV)]
                l1 = [combine(vs[2 * k], idx_c[2 * k], vs[2 * k + 1],
                              idx_c[2 * k + 1]) for k in range(4)]
                l2 = [combine(*l1[0], *l1[1]), combine(*l1[2], *l1[3])]
                bv, bi = combine(*l2[0], *l2[1])
                row_idx = iota + (sb + u * (L + 1))
                plsc.store_scatter(bvs_v, [row_idx], bv)
                plsc.store_scatter(bis_v, [row_idx], bi)
            cols_v = [plsc.load_gather(bvs_v, [col1 + (sb + p)])
                      for p in range(L)]
            cols_i = [plsc.load_gather(bis_v, [col1 + (sb + p)])
                      for p in range(L)]
            mx = cols_v
            while len(mx) > 1:
                mx = [jnp.maximum(mx[2 * i], mx[2 * i + 1])
                      for i in range(len(mx) // 2)]
            cand = [jnp.where(cols_v[p] == mx[0], cols_i[p], -1)
                    for p in range(L)]
            while len(cand) > 1:
                cand = [jnp.maximum(cand[2 * i], cand[2 * i + 1])
                        for i in range(len(cand) // 2)]
            gidx = (K - 1) - cand[0]
            valid = (t0 + base) + iota < lvb
            res_v[par, pl.ds(base, L)] = jnp.where(valid, gidx, 0)

    def start_in(b, t0, par, sem):
        pltpu.async_copy(unaries_hbm.at[b, pl.ds(t0, C)], buf2_v.at[par], sem)

    def wait_in(par, sem):
        pltpu.make_async_copy(unaries_hbm.at[0, pl.ds(0, C)], buf2_v.at[par],
                              sem).wait()

    def start_out(b, t0, par, sem):
        pltpu.async_copy(res_v.at[par], out_hbm.at[b, pl.ds(t0, C)], sem)

    def wait_out(par, sem):
        pltpu.make_async_copy(res_v.at[par], out_hbm.at[0, pl.ds(0, C)],
                              sem).wait()

    lw = plsc.load_gather(len_v, [r0 + jnp.minimum(iota, 1)])
    nzs = []
    for r in range(2):
        lnr = jnp.minimum(jnp.maximum(lw[r], 0), T)
        nchr = (lnr + (C - 1)) // C

        def zbody(c, _):
            pltpu.async_copy(out_z, out_hbm.at[r0 + r, pl.ds(c * C, C)],
                             sem_z)
            return 0

        lax.fori_loop(nchr, NCH, zbody, 0)
        nzs.append(NCH - nchr)

    ntot = jnp.maximum(ptot - wid + (NW - 1), 0) // NW

    @pl.when(ntot > 0)
    def _():
        b0 = advance(jnp.int32(0), wid)
        t0 = (wid - pgather(jnp.broadcast_to(b0, (L,)))[0]) * C
        start_in(b0, t0, 0, sem_i0)

        def chunk_body(i, b_cur):
            g = wid + NW * i
            par = i % 2
            e_b = pgather(jnp.broadcast_to(b_cur, (L,)))[0]
            t0 = (g - e_b) * C

            @pl.when(par == 0)
            def _():
                wait_in(0, sem_i0)

            @pl.when(par == 1)
            def _():
                wait_in(1, sem_i1)

            b_next = advance(b_cur, g + NW)

            @pl.when(i + 1 < ntot)
            def _():
                e_n = pgather(jnp.broadcast_to(b_next, (L,)))[0]
                t0n = (g + NW - e_n) * C

                @pl.when(par == 0)
                def _():
                    start_in(b_next, t0n, 1, sem_i1)

                @pl.when(par == 1)
                def _():
                    start_in(b_next, t0n, 0, sem_i0)

            @pl.when(i >= 2)
            def _():
                @pl.when(par == 0)
                def _():
                    wait_out(0, sem_o0)

                @pl.when(par == 1)
                def _():
                    wait_out(1, sem_o1)

            lvb = plsc.load_gather(len_v, [jnp.broadcast_to(b_cur, (L,))])
            compute_chunk(par, t0, lvb)

            @pl.when(par == 0)
            def _():
                start_out(b_cur, t0, 0, sem_o0)

            @pl.when(par == 1)
            def _():
                start_out(b_cur, t0, 1, sem_o1)

            return b_next

        lax.fori_loop(0, ntot, chunk_body, b0)

    @pl.when(ntot >= 2)
    def _():
        @pl.when(ntot % 2 == 0)
        def _():
            wait_out(0, sem_o0)

        @pl.when(ntot % 2 == 1)
        def _():
            wait_out(1, sem_o1)

    @pl.when(ntot >= 1)
    def _():
        @pl.when((ntot - 1) % 2 == 0)
        def _():
            wait_out(0, sem_o0)

        @pl.when((ntot - 1) % 2 == 1)
        def _():
            wait_out(1, sem_o1)

    def zdrain(c, _):
        pltpu.make_async_copy(out_z, out_hbm.at[0, pl.ds(0, C)], sem_z).wait()
        return 0

    lax.fori_loop(0, nzs[0] + nzs[1], zdrain, 0)


@jax.jit
def kernel(unaries, lengths):
    mesh = plsc.VectorSubcoreMesh(core_axis_name="c", subcore_axis_name="s",
                                  num_cores=NC, num_subcores=NS)
    return pl.kernel(
        _sc_body,
        out_type=jax.ShapeDtypeStruct((B, T), jnp.int32),
        mesh=mesh,
        compiler_params=pltpu.CompilerParams(needs_layout_passes=False),
        scratch_types=[
            pltpu.VMEM((B,), jnp.int32),
            pltpu.VMEM((2, C, K), jnp.float32),
            pltpu.VMEM((2, C), jnp.int32),
            pltpu.VMEM((C,), jnp.int32),
            pltpu.VMEM((C * (L + 1),), jnp.float32),
            pltpu.VMEM((C * (L + 1),), jnp.int32),
            pltpu.VMEM((B + 2 * L,), jnp.int32),
            pltpu.SemaphoreType.DMA,
            pltpu.SemaphoreType.DMA,
            pltpu.SemaphoreType.DMA,
            pltpu.SemaphoreType.DMA,
            pltpu.SemaphoreType.DMA,
        ],
    )(unaries, lengths)


# speedup vs baseline: 1.8957x; 1.0591x over previous
import jax
import jax.numpy as jnp
from jax import lax
from jax.experimental import pallas as pl
from jax.experimental.pallas import tpu as pltpu
from jax.experimental.pallas import tpu_sc as plsc

B, T, K = 64, 2048, 128
NC, NS = 2, 16
NW = NC * NS
C = 128
NCH = T // C
L = 16
KV = K // L
NBUF = 4
BIG = 2**31 - 1


def _sc_body(unaries_hbm, lengths_hbm, out_hbm, len_v, buf2_v, res_v, out_z,
             bvs_v, bis_v, pfx_v, sem_i0, sem_i1, sem_i2, sem_i3,
             sem_o0, sem_o1, sem_o2, sem_o3, sem_z):
    cid = lax.axis_index("c")
    sid = lax.axis_index("s")
    wid = sid * NC + cid
    r0 = wid * (B // NW)

    iota = lax.iota(jnp.int32, L)
    idx_c = [(K - 1 - j * L) - iota for j in range(KV)]
    col1 = iota * (L + 1)

    pltpu.sync_copy(lengths_hbm, len_v)
    zeros16 = jnp.zeros((L,), jnp.int32)
    for g in range(NCH // 2):
        out_z[pl.ds(g * L, L)] = zeros16
    carry = jnp.int32(0)
    nch_row = []
    for k in range(B // L):
        ln16 = len_v[pl.ds(k * L, L)]
        ln16 = jnp.minimum(jnp.maximum(ln16, 0), T)
        nb = (ln16 + (C - 1)) // C
        nch_row.append(nb)
        cum = plsc.cumsum(nb)
        pfx_v[pl.ds(k * L, L)] = (cum - nb) + carry
        carry = carry + cum[L - 1]
    ptot = carry
    pfx_v[pl.ds(B, L)] = jnp.where(iota == 0, ptot, BIG)
    pfx_v[pl.ds(B + L, L)] = jnp.full((L,), BIG, jnp.int32)

    def pgather(idx):
        return plsc.load_gather(pfx_v, [idx])

    def advance(b, g):
        def cond(bb):
            w16 = pgather(bb + 1 + iota) <= g
            return plsc.all_reduce_population_count(w16)[0] == L

        b = lax.while_loop(cond, lambda bb: bb + L, b)
        w16 = pgather(b + 1 + iota) <= g
        return b + plsc.all_reduce_population_count(w16)[0]

    def combine(av, ai, bv, bi):
        m = bv > av
        return jnp.where(m, bv, av), jnp.where(m, bi, ai)

    def compute_chunk(par, t0, lvb):
        @plsc.parallel_loop(0, C // L, unroll=2)
        def grp(g):
            base = g * L
            sb = g * (L * (L + 1))
            for u in range(L):
                t = base + u
                vs = [buf2_v[par, t, pl.ds(k * L, L)] for k in range(KV)]
                l1 = [combine(vs[2 * k], idx_c[2 * k], vs[2 * k + 1],
                              idx_c[2 * k + 1]) for k in range(4)]
                l2 = [combine(*l1[0], *l1[1]), combine(*l1[2], *l1[3])]
                bv, bi = combine(*l2[0], *l2[1])
                row_idx = iota + (sb + u * (L + 1))
                plsc.store_scatter(bvs_v, [row_idx], bv)
                plsc.store_scatter(bis_v, [row_idx], bi)
            cols_v = [plsc.load_gather(bvs_v, [col1 + (sb + p)])
                      for p in range(L)]
            cols_i = [plsc.load_gather(bis_v, [col1 + (sb + p)])
                      for p in range(L)]
            mx = cols_v
            while len(mx) > 1:
                mx = [jnp.maximum(mx[2 * i], mx[2 * i + 1])
                      for i in range(len(mx) // 2)]
            cand = [jnp.where(cols_v[p] == mx[0], cols_i[p], -1)
                    for p in range(L)]
            while len(cand) > 1:
                cand = [jnp.maximum(cand[2 * i], cand[2 * i + 1])
                        for i in range(len(cand) // 2)]
            gidx = (K - 1) - cand[0]
            valid = (t0 + base) + iota < lvb
            res_v[par, pl.ds(base, L)] = jnp.where(valid, gidx, 0)

    def start_in(b, t0, par, sem):
        pltpu.async_copy(unaries_hbm.at[b, pl.ds(t0, C)], buf2_v.at[par], sem)

    def wait_in(par, sem):
        pltpu.make_async_copy(unaries_hbm.at[0, pl.ds(0, C)], buf2_v.at[par],
                              sem).wait()

    def start_out(b, t0, par, sem):
        pltpu.async_copy(res_v.at[par], out_hbm.at[b, pl.ds(t0, C)], sem)

    def wait_out(par, sem):
        pltpu.make_async_copy(res_v.at[par], out_hbm.at[0, pl.ds(0, C)],
                              sem).wait()

    lw = plsc.load_gather(len_v, [r0 + jnp.minimum(iota, 1)])
    nzs = []
    for r in range(2):
        lnr = jnp.minimum(jnp.maximum(lw[r], 0), T)
        nchr = (lnr + (C - 1)) // C

        def zbody(c, _):
            pltpu.async_copy(out_z, out_hbm.at[r0 + r, pl.ds(c * C, C)],
                             sem_z)
            return 0

        lax.fori_loop(nchr, NCH, zbody, 0)
        nzs.append(NCH - nchr)

    ntot = jnp.maximum(ptot - wid + (NW - 1), 0) // NW
    sem_i = [sem_i0, sem_i1, sem_i2, sem_i3]
    sem_o = [sem_o0, sem_o1, sem_o2, sem_o3]

    def chunk_t0(b, g):
        return (g - pgather(jnp.broadcast_to(b, (L,)))[0]) * C

    @pl.when(ntot > 0)
    def _():
        b_prev = jnp.int32(0)
        bs = []
        for q in range(NBUF - 1):
            b_q = advance(b_prev, wid + NW * q)
            bs.append(b_q)
            b_prev = b_q

            @pl.when(q < ntot)
            def _(b_q=b_q, q=q):
                start_in(b_q, chunk_t0(b_q, wid + NW * q), q, sem_i[q])

        def chunk_body(i, carry):
            b_cur, b_pf = carry
            g = wid + NW * i
            par = i % NBUF
            t0 = chunk_t0(b_cur, g)

            for q in range(NBUF):
                @pl.when(par == q)
                def _(q=q):
                    wait_in(q, sem_i[q])

            b_pf2 = advance(b_pf, g + (NBUF - 1) * NW)

            @pl.when(i + (NBUF - 1) < ntot)
            def _():
                t0n = chunk_t0(b_pf2, g + (NBUF - 1) * NW)
                for q in range(NBUF):
                    @pl.when(par == (q + 1) % NBUF)
                    def _(q=q):
                        start_in(b_pf2, t0n, q, sem_i[q])

            @pl.when(i >= NBUF)
            def _():
                for q in range(NBUF):
                    @pl.when(par == q)
                    def _(q=q):
                        wait_out(q, sem_o[q])

            lvb = plsc.load_gather(len_v, [jnp.broadcast_to(b_cur, (L,))])
            compute_chunk(par, t0, lvb)

            for q in range(NBUF):
                @pl.when(par == q)
                def _(q=q):
                    start_out(b_cur, t0, q, sem_o[q])

            b_nxt = advance(b_cur, g + NW)
            return (b_nxt, b_pf2)

        lax.fori_loop(0, ntot, chunk_body, (bs[0], bs[NBUF - 2]))

    for q in range(NBUF):
        @pl.when(ntot > q)
        def _(q=q):
            for m in range(NBUF):
                @pl.when((ntot - 1 - q) % NBUF == m)
                def _(m=m):
                    wait_out(m, sem_o[m])

    def zdrain(c, _):
        pltpu.make_async_copy(out_z, out_hbm.at[0, pl.ds(0, C)], sem_z).wait()
        return 0

    lax.fori_loop(0, nzs[0] + nzs[1], zdrain, 0)


@jax.jit
def kernel(unaries, lengths):
    mesh = plsc.VectorSubcoreMesh(core_axis_name="c", subcore_axis_name="s",
                                  num_cores=NC, num_subcores=NS)
    return pl.kernel(
        _sc_body,
        out_type=jax.ShapeDtypeStruct((B, T), jnp.int32),
        mesh=mesh,
        compiler_params=pltpu.CompilerParams(needs_layout_passes=False),
        scratch_types=[
            pltpu.VMEM((B,), jnp.int32),
            pltpu.VMEM((NBUF, C, K), jnp.float32),
            pltpu.VMEM((NBUF, C), jnp.int32),
            pltpu.VMEM((C,), jnp.int32),
            pltpu.VMEM((C * (L + 1),), jnp.float32),
            pltpu.VMEM((C * (L + 1),), jnp.int32),
            pltpu.VMEM((B + 2 * L,), jnp.int32),
        ] + [pltpu.SemaphoreType.DMA] * 9,
    )(unaries, lengths)
